# Initial kernel scaffold; baseline (speedup 1.0000x reference)
import jax
import jax.numpy as jnp
from jax.experimental import pallas as pl


def kernel(x, table):
    raise NotImplementedError("write your pallas kernel here")


# speedup vs baseline: 4.7319x; 4.7319x over previous
import functools
import math

import jax
import jax.numpy as jnp
from jax import lax
from jax.experimental import pallas as pl
from jax.experimental.pallas import tpu as pltpu
from jax.experimental.pallas import tpu_sc as plsc

D_MODEL = 128
SCALE = math.sqrt(float(D_MODEL))

_info = plsc.get_sparse_core_info()
_NC = _info.num_cores
_NS = _info.num_subcores
_NW = _NC * _NS
_L = _info.num_lanes

CHUNK = 128


@functools.lru_cache(maxsize=None)
def _build(B, V, D):
    assert B % (_NW * CHUNK) == 0
    b_per_w = B // _NW
    n_chunks = b_per_w // CHUNK
    mesh = plsc.VectorSubcoreMesh(core_axis_name="c", subcore_axis_name="s")

    @functools.partial(
        pl.kernel,
        mesh=mesh,
        out_type=jax.ShapeDtypeStruct((B, D), jnp.float32),
        scratch_types=[
            pltpu.VMEM((b_per_w,), jnp.int32),
            pltpu.VMEM((CHUNK, D), jnp.float32),
            pltpu.SemaphoreType.DMA,
        ],
    )
    def emb_kernel(idx_hbm, table_hbm, out_hbm, idx_v, rows_v, sem):
        wid = lax.axis_index("s") * _NC + lax.axis_index("c")
        base = wid * b_per_w
        pltpu.sync_copy(idx_hbm.at[pl.ds(base, b_per_w)], idx_v)
        scale_vec = jnp.full((_L,), SCALE, dtype=jnp.float32)

        def chunk_body(ci, carry):
            off = ci * CHUNK
            pltpu.async_copy(
                table_hbm.at[idx_v.at[pl.ds(off, CHUNK)]], rows_v, sem
            ).wait()

            def row_body(r, c2):
                for j in range(D // _L):
                    s = rows_v[r, pl.ds(j * _L, _L)]
                    rows_v[r, pl.ds(j * _L, _L)] = s * scale_vec
                return c2

            lax.fori_loop(0, CHUNK, row_body, 0)
            pltpu.sync_copy(rows_v, out_hbm.at[pl.ds(base + off, CHUNK)])
            return carry

        lax.fori_loop(0, n_chunks, chunk_body, 0)

    return emb_kernel


def kernel(x, table):
    B = x.shape[0] * x.shape[1]
    V, D = table.shape
    idx = x.reshape(-1).astype(jnp.int32)
    out = _build(B, V, D)(idx, table)
    return out.reshape(x.shape + (D,))


# speedup vs baseline: 7.8568x; 1.6604x over previous
import functools
import math

import jax
import jax.numpy as jnp
from jax import lax
from jax.experimental import pallas as pl
from jax.experimental.pallas import tpu as pltpu
from jax.experimental.pallas import tpu_sc as plsc

D_MODEL = 128
SCALE = math.sqrt(float(D_MODEL))

_info = plsc.get_sparse_core_info()
_NC = _info.num_cores
_NS = _info.num_subcores
_NW = _NC * _NS
_L = _info.num_lanes

CHUNK = 128


@functools.lru_cache(maxsize=None)
def _build(B, V, D):
    assert B % (_NW * CHUNK) == 0
    b_per_w = B // _NW
    n_chunks = b_per_w // CHUNK
    mesh = plsc.VectorSubcoreMesh(core_axis_name="c", subcore_axis_name="s")

    assert n_chunks % 2 == 0

    @functools.partial(
        pl.kernel,
        mesh=mesh,
        out_type=jax.ShapeDtypeStruct((B, D), jnp.float32),
        scratch_types=[
            pltpu.VMEM((b_per_w,), jnp.int32),
            pltpu.SemaphoreType.DMA,
            pltpu.VMEM((CHUNK, D), jnp.float32),
            pltpu.VMEM((CHUNK, D), jnp.float32),
            pltpu.VMEM((CHUNK, D), jnp.float32),
            pltpu.VMEM((CHUNK, D), jnp.float32),
            pltpu.SemaphoreType.DMA,
            pltpu.SemaphoreType.DMA,
            pltpu.SemaphoreType.DMA,
            pltpu.SemaphoreType.DMA,
        ],
    )
    def emb_kernel(idx_hbm, table_hbm, out_hbm,
                   idx_v, isem, g0, g1, o0, o1, gs0, gs1, os0, os1):
        wid = lax.axis_index("s") * _NC + lax.axis_index("c")
        base = wid * b_per_w
        pltpu.async_copy(idx_hbm.at[pl.ds(base, b_per_w)], idx_v, isem).wait()
        gbuf = (g0, g1)
        obuf = (o0, o1)
        gsem = (gs0, gs1)
        osem = (os0, os1)
        scale_vec = jnp.full((_L,), SCALE, dtype=jnp.float32)

        def gather(ci, b):
            return pltpu.async_copy(
                table_hbm.at[idx_v.at[pl.ds(ci * CHUNK, CHUNK)]],
                gbuf[b], gsem[b])

        gather(0, 0)
        gather(1, 1)

        def outer(ci0, carry):
            for b in range(2):
                ci = ci0 + b
                src, dst = gbuf[b], obuf[b]
                pltpu.make_async_copy(out_hbm.at[pl.ds(0, CHUNK)],
                                      src, gsem[b]).wait()
                @pl.when(ci0 > 0)
                def _():
                    pltpu.make_async_copy(dst, out_hbm.at[pl.ds(0, CHUNK)],
                                          osem[b]).wait()

                def row_body(r, c2):
                    for j in range(D // _L):
                        s = src[r, pl.ds(j * _L, _L)]
                        dst[r, pl.ds(j * _L, _L)] = s * scale_vec
                    return c2

                lax.fori_loop(0, CHUNK, row_body, 0)

                @pl.when(ci + 2 < n_chunks)
                def _():
                    gather(ci + 2, b)

                pltpu.async_copy(dst, out_hbm.at[pl.ds(base + ci * CHUNK, CHUNK)],
                                 osem[b])
            return carry

        lax.fori_loop(0, n_chunks // 2, lambda i, c: outer(i * 2, c), 0)
        for b in range(2):
            pltpu.make_async_copy(obuf[b], out_hbm.at[pl.ds(0, CHUNK)],
                                  osem[b]).wait()

    return emb_kernel


def kernel(x, table):
    B = x.shape[0] * x.shape[1]
    V, D = table.shape
    idx = x.reshape(-1).astype(jnp.int32)
    out = _build(B, V, D)(idx, table)
    return out.reshape(x.shape + (D,))


# speedup vs baseline: 7.8872x; 1.0039x over previous
import functools
import math

import jax
import jax.numpy as jnp
from jax import lax
from jax.experimental import pallas as pl
from jax.experimental.pallas import tpu as pltpu
from jax.experimental.pallas import tpu_sc as plsc

D_MODEL = 128
SCALE = math.sqrt(float(D_MODEL))

_info = plsc.get_sparse_core_info()
_NC = _info.num_cores
_NS = _info.num_subcores
_NW = _NC * _NS
_L = _info.num_lanes

CHUNK = 128
NG = 3
NO = 2
UNROLL = 6


@functools.lru_cache(maxsize=None)
def _build(B, V, D):
    assert B % (_NW * CHUNK) == 0
    b_per_w = B // _NW
    n_chunks = b_per_w // CHUNK
    n_main = (n_chunks // UNROLL) * UNROLL
    mesh = plsc.VectorSubcoreMesh(core_axis_name="c", subcore_axis_name="s")

    @functools.partial(
        pl.kernel,
        mesh=mesh,
        out_type=jax.ShapeDtypeStruct((B, D), jnp.float32),
        scratch_types=[
            pltpu.VMEM((b_per_w,), jnp.int32),
            pltpu.SemaphoreType.DMA,
        ] + [pltpu.VMEM((CHUNK, D), jnp.float32)] * (NG + NO)
          + [pltpu.SemaphoreType.DMA] * (NG + NO),
    )
    def emb_kernel(idx_hbm, table_hbm, out_hbm, idx_v, isem, *bufs_and_sems):
        gbuf = bufs_and_sems[:NG]
        obuf = bufs_and_sems[NG:NG + NO]
        gsem = bufs_and_sems[NG + NO:2 * NG + NO]
        osem = bufs_and_sems[2 * NG + NO:]
        wid = lax.axis_index("s") * _NC + lax.axis_index("c")
        base = wid * b_per_w
        pltpu.async_copy(idx_hbm.at[pl.ds(base, b_per_w)], idx_v, isem).wait()
        scale_vec = jnp.full((_L,), SCALE, dtype=jnp.float32)

        def gather(ci, g):
            pltpu.async_copy(
                table_hbm.at[idx_v.at[pl.ds(ci * CHUNK, CHUNK)]],
                gbuf[g], gsem[g])

        def wait_gather(g):
            pltpu.make_async_copy(out_hbm.at[pl.ds(0, CHUNK)],
                                  gbuf[g], gsem[g]).wait()

        def wait_out(o):
            pltpu.make_async_copy(obuf[o], out_hbm.at[pl.ds(0, CHUNK)],
                                  osem[o]).wait()

        def scale(g, o):
            src, dst = gbuf[g], obuf[o]

            def row_body(r2, c2):
                for rr in range(2):
                    r = r2 * 2 + rr
                    for j in range(D // _L):
                        dst[r, pl.ds(j * _L, _L)] = (
                            src[r, pl.ds(j * _L, _L)] * scale_vec)
                return c2

            lax.fori_loop(0, CHUNK // 2, row_body, 0)

        def step(ci, k, first, fire):
            g, o = k % NG, k % NO
            wait_gather(g)
            if fire:
                gather(ci + NG - 1, (k + NG - 1) % NG)
            if not first:
                wait_out(o)
            scale(g, o)
            pltpu.async_copy(obuf[o],
                             out_hbm.at[pl.ds(base + ci * CHUNK, CHUNK)],
                             osem[o])

        for ci in range(NG - 1):
            gather(ci, ci % NG)

        def outer(i, carry):
            ci0 = i * UNROLL
            for k in range(UNROLL):
                step(ci0 + k, k, first=False, fire=True)
            return carry

        for ci in range(UNROLL):
            step(ci, ci, first=ci < NO, fire=True)
        lax.fori_loop(1, n_main // UNROLL, outer, 0)
        for ci in range(n_main, n_chunks):
            step(ci, ci % UNROLL, first=False, fire=False)

        for o in range(NO):
            wait_out(o)

    return emb_kernel


def kernel(x, table):
    B = x.shape[0] * x.shape[1]
    V, D = table.shape
    idx = x.reshape(-1).astype(jnp.int32)
    out = _build(B, V, D)(idx, table)
    return out.reshape(x.shape + (D,))
